# baseline (speedup 1.0000x reference)
import functools

import jax
import jax.numpy as jnp
from jax import lax
from jax.experimental import pallas as pl
from jax.experimental.pallas import tpu as pltpu
from jax.experimental.pallas import tpu_sc as plsc

_TOPK = 4
_ROWS = 1024
_COLS = 32768
_NC = 2
_NS = 16
_L = 16
_NW = _NC * _NS
_RPW = _ROWS // _NW
_GROUPS = _RPW // 4
_GRP = 8
_STEPS = _COLS // (_L * _GRP)
_NEG_INF = float("-inf")


def _insert(v, idx, t0, t1, t2, t3, i0, i1, i2, i3):
    m0 = v > t0
    m1 = v > t1
    m2 = v > t2
    m3 = v > t3
    t3n = jnp.where(m2, t2, jnp.where(m3, v, t3))
    i3n = jnp.where(m2, i2, jnp.where(m3, idx, i3))
    t2n = jnp.where(m1, t1, jnp.where(m2, v, t2))
    i2n = jnp.where(m1, i1, jnp.where(m2, idx, i2))
    t1n = jnp.where(m0, t0, jnp.where(m1, v, t1))
    i1n = jnp.where(m0, i0, jnp.where(m1, idx, i1))
    t0n = jnp.where(m0, v, t0)
    i0n = jnp.where(m0, idx, i0)
    return t0n, t1n, t2n, t3n, i0n, i1n, i2n, i3n


def _scan_row(buf, iota):
    tneg = jnp.full((_L,), _NEG_INF, jnp.float32)
    izero = jnp.zeros((_L,), jnp.int32)

    def body(c, carry):
        base = c * (_L * _GRP)
        vs = [buf[pl.ds(base + u * _L, _L)] for u in range(_GRP)]
        mx = vs[0]
        for u in range(1, _GRP):
            mx = jnp.maximum(mx, vs[u])
        hit = jnp.any(mx > carry[3])

        def do_insert(carry):
            t0, t1, t2, t3, i0, i1, i2, i3 = carry
            for u in range(_GRP):
                idx = iota + (base + u * _L)
                t0, t1, t2, t3, i0, i1, i2, i3 = _insert(
                    vs[u], idx, t0, t1, t2, t3, i0, i1, i2, i3)
            return (t0, t1, t2, t3, i0, i1, i2, i3)

        return lax.cond(hit, do_insert, lambda carry: carry, carry)

    init = (tneg, tneg, tneg, tneg, izero, izero, izero, izero)
    return lax.fori_loop(0, _STEPS, body, init)


def _merge_softmax(carry, lane_off, iota):
    ts = list(carry[:4])
    is_ = list(carry[4:])
    big = jnp.int32(2**30)
    gv, gi = [], []
    for _ in range(_TOPK):
        m = jnp.maximum(jnp.maximum(ts[0], ts[1]), jnp.maximum(ts[2], ts[3]))
        gmax = jnp.max(m)
        cand = [jnp.where(tj == gmax, ij, big) for tj, ij in zip(ts, is_)]
        mn = jnp.minimum(jnp.minimum(cand[0], cand[1]),
                         jnp.minimum(cand[2], cand[3]))
        gidx = jnp.min(mn)
        gv.append(gmax)
        gi.append(gidx)
        ts = [jnp.where(ij == gidx, _NEG_INF, tj) for tj, ij in zip(ts, is_)]
    dv = jnp.zeros((_L,), jnp.float32)
    iv = jnp.zeros((_L,), jnp.int32)
    for k in range(_TOPK):
        sel = iota == (lane_off + k)
        dv = jnp.where(sel, gv[k] - gv[0], dv)
        iv = jnp.where(sel, gi[k], iv)
    ev = jnp.exp(dv)
    in_row = (iota >= lane_off) & (iota < lane_off + _TOPK)
    ev = jnp.where(in_row, ev, 0.0)
    wv = ev / jnp.sum(ev)
    return wv, iv


def _make_kernel():
    mesh = plsc.VectorSubcoreMesh(core_axis_name="c", subcore_axis_name="s",
                                  num_cores=_NC, num_subcores=_NS)

    @functools.partial(
        pl.kernel,
        out_type=(
            jax.ShapeDtypeStruct((_ROWS * _TOPK,), jnp.float32),
            jax.ShapeDtypeStruct((_ROWS * _TOPK,), jnp.int32),
        ),
        mesh=mesh,
        scratch_types=(
            pltpu.VMEM((_COLS,), jnp.float32),
            pltpu.VMEM((_COLS,), jnp.float32),
            pltpu.VMEM((_RPW * _TOPK,), jnp.float32),
            pltpu.VMEM((_RPW * _TOPK,), jnp.int32),
            pltpu.SemaphoreType.DMA,
        ),
        compiler_params=pltpu.CompilerParams(needs_layout_passes=False),
    )
    def topk_route(adj_hbm, out_w_hbm, out_i_hbm, buf0, buf1, stw, sti, sem):
        cid = lax.axis_index("c")
        sid = lax.axis_index("s")
        wid = sid * _NC + cid
        row0 = wid * _RPW
        iota = lax.iota(jnp.int32, _L)

        def row_slice(r):
            return adj_hbm.at[pl.ds(r * _COLS, _COLS)]

        pltpu.sync_copy(row_slice(row0), buf0)

        def group(g, acc):
            r0 = row0 + 4 * g
            pltpu.async_copy(row_slice(r0 + 1), buf1, sem)
            w0, j0 = _merge_softmax(_scan_row(buf0, iota), 0, iota)
            pltpu.make_async_copy(row_slice(r0 + 1), buf1, sem).wait()

            pltpu.async_copy(row_slice(r0 + 2), buf0, sem)
            w1, j1 = _merge_softmax(_scan_row(buf1, iota), 4, iota)
            pltpu.make_async_copy(row_slice(r0 + 2), buf0, sem).wait()

            pltpu.async_copy(row_slice(r0 + 3), buf1, sem)
            w2, j2 = _merge_softmax(_scan_row(buf0, iota), 8, iota)
            pltpu.make_async_copy(row_slice(r0 + 3), buf1, sem).wait()

            @pl.when(g < _GROUPS - 1)
            def _():
                pltpu.async_copy(row_slice(r0 + 4), buf0, sem)

            w3, j3 = _merge_softmax(_scan_row(buf1, iota), 12, iota)

            @pl.when(g < _GROUPS - 1)
            def _():
                pltpu.make_async_copy(row_slice(r0 + 4), buf0, sem).wait()

            stw[pl.ds(g * _L, _L)] = w0 + w1 + w2 + w3
            sti[pl.ds(g * _L, _L)] = j0 + j1 + j2 + j3
            return acc

        lax.fori_loop(0, _GROUPS, group, jnp.int32(0))

        pltpu.sync_copy(stw, out_w_hbm.at[pl.ds(row0 * _TOPK, _RPW * _TOPK)])
        pltpu.sync_copy(sti, out_i_hbm.at[pl.ds(row0 * _TOPK, _RPW * _TOPK)])

    return topk_route


_topk_route = _make_kernel()


@jax.jit
def kernel(adj):
    b, h, n = adj.shape
    flat = adj.reshape(b * h * n)
    w, i = _topk_route(flat)
    return w.reshape(b, h, _TOPK), i.reshape(b, h, _TOPK)


# speedup vs baseline: 1.1501x; 1.1501x over previous
import functools

import jax
import jax.numpy as jnp
from jax import lax
from jax.experimental import pallas as pl
from jax.experimental.pallas import tpu as pltpu
from jax.experimental.pallas import tpu_sc as plsc

_TOPK = 4
_ROWS = 1024
_COLS = 32768
_NC = 2
_NS = 16
_L = 16
_NW = _NC * _NS
_RPW = _ROWS // _NW
_GROUPS = _RPW // 4
_GRP = 32
_STEPS = _COLS // (_L * _GRP)
_NEG_INF = float("-inf")


def _insert(v, idx, t0, t1, t2, t3, i0, i1, i2, i3):
    m0 = v > t0
    m1 = v > t1
    m2 = v > t2
    m3 = v > t3
    t3n = jnp.where(m2, t2, jnp.where(m3, v, t3))
    i3n = jnp.where(m2, i2, jnp.where(m3, idx, i3))
    t2n = jnp.where(m1, t1, jnp.where(m2, v, t2))
    i2n = jnp.where(m1, i1, jnp.where(m2, idx, i2))
    t1n = jnp.where(m0, t0, jnp.where(m1, v, t1))
    i1n = jnp.where(m0, i0, jnp.where(m1, idx, i1))
    t0n = jnp.where(m0, v, t0)
    i0n = jnp.where(m0, idx, i0)
    return t0n, t1n, t2n, t3n, i0n, i1n, i2n, i3n


def _scan_row(buf, iota):
    tneg = jnp.full((_L,), _NEG_INF, jnp.float32)
    izero = jnp.zeros((_L,), jnp.int32)

    def body(c, carry):
        base = c * (_L * _GRP)
        mx = buf[pl.ds(base, _L)]
        for u in range(1, _GRP):
            mx = jnp.maximum(mx, buf[pl.ds(base + u * _L, _L)])
        hit = jnp.any(mx > carry[3])

        def do_insert(carry):
            t0, t1, t2, t3, i0, i1, i2, i3 = carry
            for u in range(_GRP):
                off = base + u * _L
                v = buf[pl.ds(off, _L)]
                idx = iota + off
                t0, t1, t2, t3, i0, i1, i2, i3 = _insert(
                    v, idx, t0, t1, t2, t3, i0, i1, i2, i3)
            return (t0, t1, t2, t3, i0, i1, i2, i3)

        return lax.cond(hit, do_insert, lambda carry: carry, carry)

    init = (tneg, tneg, tneg, tneg, izero, izero, izero, izero)
    return lax.fori_loop(0, _STEPS, body, init)


def _merge_softmax(carry, lane_off, iota):
    ts = list(carry[:4])
    is_ = list(carry[4:])
    big = jnp.int32(2**30)
    gv, gi = [], []
    for _ in range(_TOPK):
        m = jnp.maximum(jnp.maximum(ts[0], ts[1]), jnp.maximum(ts[2], ts[3]))
        gmax = jnp.max(m)
        cand = [jnp.where(tj == gmax, ij, big) for tj, ij in zip(ts, is_)]
        mn = jnp.minimum(jnp.minimum(cand[0], cand[1]),
                         jnp.minimum(cand[2], cand[3]))
        gidx = jnp.min(mn)
        gv.append(gmax)
        gi.append(gidx)
        ts = [jnp.where(ij == gidx, _NEG_INF, tj) for tj, ij in zip(ts, is_)]
    dv = jnp.zeros((_L,), jnp.float32)
    iv = jnp.zeros((_L,), jnp.int32)
    for k in range(_TOPK):
        sel = iota == (lane_off + k)
        dv = jnp.where(sel, gv[k] - gv[0], dv)
        iv = jnp.where(sel, gi[k], iv)
    ev = jnp.exp(dv)
    in_row = (iota >= lane_off) & (iota < lane_off + _TOPK)
    ev = jnp.where(in_row, ev, 0.0)
    wv = ev / jnp.sum(ev)
    return wv, iv


def _make_kernel():
    mesh = plsc.VectorSubcoreMesh(core_axis_name="c", subcore_axis_name="s",
                                  num_cores=_NC, num_subcores=_NS)

    @functools.partial(
        pl.kernel,
        out_type=(
            jax.ShapeDtypeStruct((_ROWS * _TOPK,), jnp.float32),
            jax.ShapeDtypeStruct((_ROWS * _TOPK,), jnp.int32),
        ),
        mesh=mesh,
        scratch_types=(
            pltpu.VMEM((_COLS,), jnp.float32),
            pltpu.VMEM((_COLS,), jnp.float32),
            pltpu.VMEM((_RPW * _TOPK,), jnp.float32),
            pltpu.VMEM((_RPW * _TOPK,), jnp.int32),
            pltpu.SemaphoreType.DMA,
        ),
        compiler_params=pltpu.CompilerParams(needs_layout_passes=False),
    )
    def topk_route(adj_hbm, out_w_hbm, out_i_hbm, buf0, buf1, stw, sti, sem):
        cid = lax.axis_index("c")
        sid = lax.axis_index("s")
        wid = sid * _NC + cid
        row0 = wid * _RPW
        iota = lax.iota(jnp.int32, _L)

        def row_slice(r):
            return adj_hbm.at[pl.ds(r * _COLS, _COLS)]

        pltpu.sync_copy(row_slice(row0), buf0)

        def group(g, acc):
            r0 = row0 + 4 * g
            pltpu.async_copy(row_slice(r0 + 1), buf1, sem)
            w0, j0 = _merge_softmax(_scan_row(buf0, iota), 0, iota)
            pltpu.make_async_copy(row_slice(r0 + 1), buf1, sem).wait()

            pltpu.async_copy(row_slice(r0 + 2), buf0, sem)
            w1, j1 = _merge_softmax(_scan_row(buf1, iota), 4, iota)
            pltpu.make_async_copy(row_slice(r0 + 2), buf0, sem).wait()

            pltpu.async_copy(row_slice(r0 + 3), buf1, sem)
            w2, j2 = _merge_softmax(_scan_row(buf0, iota), 8, iota)
            pltpu.make_async_copy(row_slice(r0 + 3), buf1, sem).wait()

            @pl.when(g < _GROUPS - 1)
            def _():
                pltpu.async_copy(row_slice(r0 + 4), buf0, sem)

            w3, j3 = _merge_softmax(_scan_row(buf1, iota), 12, iota)

            @pl.when(g < _GROUPS - 1)
            def _():
                pltpu.make_async_copy(row_slice(r0 + 4), buf0, sem).wait()

            stw[pl.ds(g * _L, _L)] = w0 + w1 + w2 + w3
            sti[pl.ds(g * _L, _L)] = j0 + j1 + j2 + j3
            return acc

        lax.fori_loop(0, _GROUPS, group, jnp.int32(0))

        pltpu.sync_copy(stw, out_w_hbm.at[pl.ds(row0 * _TOPK, _RPW * _TOPK)])
        pltpu.sync_copy(sti, out_i_hbm.at[pl.ds(row0 * _TOPK, _RPW * _TOPK)])

    return topk_route


_topk_route = _make_kernel()


@jax.jit
def kernel(adj):
    b, h, n = adj.shape
    flat = adj.reshape(b * h * n)
    w, i = _topk_route(flat)
    return w.reshape(b, h, _TOPK), i.reshape(b, h, _TOPK)


# speedup vs baseline: 1.4763x; 1.2836x over previous
import functools

import jax
import jax.numpy as jnp
from jax import lax
from jax.experimental import pallas as pl
from jax.experimental.pallas import tpu as pltpu
from jax.experimental.pallas import tpu_sc as plsc

_TOPK = 4
_ROWS = 1024
_COLS = 32768
_NC = 2
_NS = 16
_L = 16
_NW = _NC * _NS
_RPW = _ROWS // _NW
_GROUPS = _RPW // 4
_GRP = 32
_STEPS = _COLS // (_L * _GRP)
_NEG_INF = float("-inf")


def _insert(v, idx, t0, t1, t2, t3, i0, i1, i2, i3):
    m0 = v > t0
    m1 = v > t1
    m2 = v > t2
    m3 = v > t3
    t3n = jnp.where(m2, t2, jnp.where(m3, v, t3))
    i3n = jnp.where(m2, i2, jnp.where(m3, idx, i3))
    t2n = jnp.where(m1, t1, jnp.where(m2, v, t2))
    i2n = jnp.where(m1, i1, jnp.where(m2, idx, i2))
    t1n = jnp.where(m0, t0, jnp.where(m1, v, t1))
    i1n = jnp.where(m0, i0, jnp.where(m1, idx, i1))
    t0n = jnp.where(m0, v, t0)
    i0n = jnp.where(m0, idx, i0)
    return t0n, t1n, t2n, t3n, i0n, i1n, i2n, i3n


def _scan_row(buf, iota):
    tneg = jnp.full((_L,), _NEG_INF, jnp.float32)
    izero = jnp.zeros((_L,), jnp.int32)

    def body(c, carry):
        base = c * (_L * _GRP)
        mx = buf[pl.ds(base, _L)]
        for u in range(1, _GRP):
            mx = jnp.maximum(mx, buf[pl.ds(base + u * _L, _L)])
        hit = jnp.any(mx > carry[3])

        def do_insert(carry):
            t0, t1, t2, t3, i0, i1, i2, i3 = carry
            for u in range(_GRP):
                off = base + u * _L
                v = buf[pl.ds(off, _L)]
                idx = iota + off
                t0, t1, t2, t3, i0, i1, i2, i3 = _insert(
                    v, idx, t0, t1, t2, t3, i0, i1, i2, i3)
            return (t0, t1, t2, t3, i0, i1, i2, i3)

        return lax.cond(hit, do_insert, lambda carry: carry, carry)

    init = (tneg, tneg, tneg, tneg, izero, izero, izero, izero)
    return lax.fori_loop(0, _STEPS, body, init)


def _merge_softmax(carry, lane_off, iota):
    ts = list(carry[:4])
    is_ = list(carry[4:])
    big = jnp.int32(2**30)
    gv, gi = [], []
    for _ in range(_TOPK):
        m = jnp.maximum(jnp.maximum(ts[0], ts[1]), jnp.maximum(ts[2], ts[3]))
        gmax = jnp.max(m)
        cand = [jnp.where(tj == gmax, ij, big) for tj, ij in zip(ts, is_)]
        mn = jnp.minimum(jnp.minimum(cand[0], cand[1]),
                         jnp.minimum(cand[2], cand[3]))
        gidx = jnp.min(mn)
        gv.append(gmax)
        gi.append(gidx)
        ts = [jnp.where(ij == gidx, _NEG_INF, tj) for tj, ij in zip(ts, is_)]
    dv = jnp.zeros((_L,), jnp.float32)
    iv = jnp.zeros((_L,), jnp.int32)
    for k in range(_TOPK):
        sel = iota == (lane_off + k)
        dv = jnp.where(sel, gv[k] - gv[0], dv)
        iv = jnp.where(sel, gi[k], iv)
    ev = jnp.exp(dv)
    in_row = (iota >= lane_off) & (iota < lane_off + _TOPK)
    ev = jnp.where(in_row, ev, 0.0)
    wv = ev / jnp.sum(ev)
    return wv, iv


def _make_kernel():
    mesh = plsc.VectorSubcoreMesh(core_axis_name="c", subcore_axis_name="s",
                                  num_cores=_NC, num_subcores=_NS)

    @functools.partial(
        pl.kernel,
        out_type=(
            jax.ShapeDtypeStruct((_ROWS * _TOPK,), jnp.float32),
            jax.ShapeDtypeStruct((_ROWS * _TOPK,), jnp.int32),
        ),
        mesh=mesh,
        scratch_types=(
            pltpu.VMEM((_COLS,), jnp.float32),
            pltpu.VMEM((_COLS,), jnp.float32),
            pltpu.VMEM((_RPW * _TOPK,), jnp.float32),
            pltpu.VMEM((_RPW * _TOPK,), jnp.int32),
            pltpu.SemaphoreType.DMA,
        ),
        compiler_params=pltpu.CompilerParams(needs_layout_passes=False),
    )
    def topk_route(adj_hbm, out_w_hbm, out_i_hbm, buf0, buf1, stw, sti, sem):
        cid = lax.axis_index("c")
        sid = lax.axis_index("s")
        wid = sid * _NC + cid
        row0 = wid * _RPW
        iota = lax.iota(jnp.int32, _L)

        def row_slice(r):
            return adj_hbm.at[r]

        pltpu.sync_copy(row_slice(row0), buf0)

        def group(g, acc):
            r0 = row0 + 4 * g
            pltpu.async_copy(row_slice(r0 + 1), buf1, sem)
            w0, j0 = _merge_softmax(_scan_row(buf0, iota), 0, iota)
            pltpu.make_async_copy(row_slice(r0 + 1), buf1, sem).wait()

            pltpu.async_copy(row_slice(r0 + 2), buf0, sem)
            w1, j1 = _merge_softmax(_scan_row(buf1, iota), 4, iota)
            pltpu.make_async_copy(row_slice(r0 + 2), buf0, sem).wait()

            pltpu.async_copy(row_slice(r0 + 3), buf1, sem)
            w2, j2 = _merge_softmax(_scan_row(buf0, iota), 8, iota)
            pltpu.make_async_copy(row_slice(r0 + 3), buf1, sem).wait()

            @pl.when(g < _GROUPS - 1)
            def _():
                pltpu.async_copy(row_slice(r0 + 4), buf0, sem)

            w3, j3 = _merge_softmax(_scan_row(buf1, iota), 12, iota)

            @pl.when(g < _GROUPS - 1)
            def _():
                pltpu.make_async_copy(row_slice(r0 + 4), buf0, sem).wait()

            stw[pl.ds(g * _L, _L)] = w0 + w1 + w2 + w3
            sti[pl.ds(g * _L, _L)] = j0 + j1 + j2 + j3
            return acc

        lax.fori_loop(0, _GROUPS, group, jnp.int32(0))

        pltpu.sync_copy(stw, out_w_hbm.at[pl.ds(row0 * _TOPK, _RPW * _TOPK)])
        pltpu.sync_copy(sti, out_i_hbm.at[pl.ds(row0 * _TOPK, _RPW * _TOPK)])

    return topk_route


_topk_route = _make_kernel()


@jax.jit
def kernel(adj):
    b, h, n = adj.shape
    w, i = _topk_route(adj.reshape(b * h, n))
    return w.reshape(b, h, _TOPK), i.reshape(b, h, _TOPK)


# speedup vs baseline: 3.9089x; 2.6478x over previous
import functools

import jax
import jax.numpy as jnp
from jax import lax
from jax.experimental import pallas as pl
from jax.experimental.pallas import tpu as pltpu
from jax.experimental.pallas import tpu_sc as plsc

_TOPK = 4
_ROWS = 1024
_COLS = 32768
_NC = 2
_NS = 16
_L = 16
_NW = _NC * _NS
_RPW = _ROWS // _NW
_GROUPS = _RPW // 4
_GRP = 32
_NGRP = _COLS // (_L * _GRP)
_SPLITS = 8
_Q = _COLS // _SPLITS
_NEG_INF = float("-inf")


def _insert(v, idx, t0, t1, t2, t3, i0, i1, i2, i3):
    m0 = v > t0
    m1 = v > t1
    m2 = v > t2
    m3 = v > t3
    t3n = jnp.where(m2, t2, jnp.where(m3, v, t3))
    i3n = jnp.where(m2, i2, jnp.where(m3, idx, i3))
    t2n = jnp.where(m1, t1, jnp.where(m2, v, t2))
    i2n = jnp.where(m1, i1, jnp.where(m2, idx, i2))
    t1n = jnp.where(m0, t0, jnp.where(m1, v, t1))
    i1n = jnp.where(m0, i0, jnp.where(m1, idx, i1))
    t0n = jnp.where(m0, v, t0)
    i0n = jnp.where(m0, idx, i0)
    return t0n, t1n, t2n, t3n, i0n, i1n, i2n, i3n


def _init_state():
    tneg = jnp.full((_L,), _NEG_INF, jnp.float32)
    izero = jnp.zeros((_L,), jnp.int32)
    return (tneg, tneg, tneg, tneg, izero, izero, izero, izero)


def _merge4(state):
    ts = list(state[:4])
    is_ = list(state[4:])
    big = jnp.int32(2**30)
    gv, gi = [], []
    for _ in range(_TOPK):
        m = jnp.maximum(jnp.maximum(ts[0], ts[1]), jnp.maximum(ts[2], ts[3]))
        gmax = jnp.max(m)
        cand = [jnp.where(tj == gmax, ij, big) for tj, ij in zip(ts, is_)]
        mn = jnp.minimum(jnp.minimum(cand[0], cand[1]),
                         jnp.minimum(cand[2], cand[3]))
        gidx = jnp.min(mn)
        gv.append(gmax)
        gi.append(gidx)
        ts = [jnp.where(ij == gidx, _NEG_INF, tj) for tj, ij in zip(ts, is_)]
    return gv, gi


def _sort4(a, b, c, d):
    a, b = jnp.minimum(a, b), jnp.maximum(a, b)
    c, d = jnp.minimum(c, d), jnp.maximum(c, d)
    a, c = jnp.minimum(a, c), jnp.maximum(a, c)
    b, d = jnp.minimum(b, d), jnp.maximum(b, d)
    b, c = jnp.minimum(b, c), jnp.maximum(b, c)
    return a, b, c, d


def _row_topk(buf, summary, gids, iota, izero):
    def phase_a(g, acc):
        base = g * (_L * _GRP)
        mx = buf[pl.ds(base, _L)]
        for u in range(1, _GRP):
            mx = jnp.maximum(mx, buf[pl.ds(base + u * _L, _L)])
        summary[pl.ds(g * _L, _L)] = mx
        return acc

    lax.fori_loop(0, _NGRP, phase_a, jnp.int32(0))

    def sum_scan(c, carry):
        t0, t1, t2, t3, i0, i1, i2, i3 = carry
        for u in range(4):
            g = c * 4 + u
            sv = summary[pl.ds(g * _L, _L)]
            t0, t1, t2, t3, i0, i1, i2, i3 = _insert(
                sv, izero + g, t0, t1, t2, t3, i0, i1, i2, i3)
        return (t0, t1, t2, t3, i0, i1, i2, i3)

    sstate = lax.fori_loop(0, _NGRP // 4, sum_scan, _init_state())
    gv, gg = _merge4(sstate)
    v4 = gv[3]

    def count_body(c, acc):
        for u in range(4):
            sv = summary[pl.ds((c * 4 + u) * _L, _L)]
            acc = acc + jnp.where(sv == v4, 1, 0)
        return acc

    cvec = lax.fori_loop(0, _NGRP // 4, count_body, jnp.zeros((_L,), jnp.int32))
    cnt = jnp.sum(cvec)
    tied = cnt > 1

    g0, g1, g2, g3 = _sort4(gg[0], gg[1], gg[2], gg[3])

    @pl.when(jnp.logical_not(tied))
    def _():
        gids[0] = g0
        gids[1] = g1
        gids[2] = g2
        gids[3] = g3

    @pl.when(tied)
    def _():
        def wr(k, acc):
            gids[k] = k
            return acc
        lax.fori_loop(0, _NGRP, wr, jnp.int32(0))

    ngrp = jnp.where(tied, _NGRP, 4)

    def per_group(k, carry):
        g = gids[k]

        def chunk8(c2, carry):
            t0, t1, t2, t3, i0, i1, i2, i3 = carry
            base = g * (_L * _GRP) + c2 * (_L * 8)
            for u in range(8):
                off = base + u * _L
                v = buf[pl.ds(off, _L)]
                t0, t1, t2, t3, i0, i1, i2, i3 = _insert(
                    v, iota + off, t0, t1, t2, t3, i0, i1, i2, i3)
            return (t0, t1, t2, t3, i0, i1, i2, i3)

        return lax.fori_loop(0, _GRP // 8, chunk8, carry)

    return lax.fori_loop(0, ngrp, per_group, _init_state())


def _softmax_pack(state, lane_off, iota):
    gv, gi = _merge4(state)
    dv = jnp.zeros((_L,), jnp.float32)
    iv = jnp.zeros((_L,), jnp.int32)
    for k in range(_TOPK):
        sel = iota == (lane_off + k)
        dv = jnp.where(sel, gv[k] - gv[0], dv)
        iv = jnp.where(sel, gi[k], iv)
    ev = jnp.exp(dv)
    in_row = (iota >= lane_off) & (iota < lane_off + _TOPK)
    ev = jnp.where(in_row, ev, 0.0)
    wv = ev / jnp.sum(ev)
    return wv, iv


def _make_kernel():
    mesh = plsc.VectorSubcoreMesh(core_axis_name="c", subcore_axis_name="s",
                                  num_cores=_NC, num_subcores=_NS)

    @functools.partial(
        pl.kernel,
        out_type=(
            jax.ShapeDtypeStruct((_ROWS * _TOPK,), jnp.float32),
            jax.ShapeDtypeStruct((_ROWS * _TOPK,), jnp.int32),
        ),
        mesh=mesh,
        scratch_types=(
            pltpu.VMEM((_COLS,), jnp.float32),
            pltpu.VMEM((_COLS,), jnp.float32),
            pltpu.VMEM((_NGRP * _L,), jnp.float32),
            pltpu.VMEM((_RPW * _TOPK,), jnp.float32),
            pltpu.VMEM((_RPW * _TOPK,), jnp.int32),
            pltpu.SMEM((_NGRP,), jnp.int32),
            pltpu.SemaphoreType.DMA,
        ),
        compiler_params=pltpu.CompilerParams(needs_layout_passes=False),
    )
    def topk_route(adj_hbm, out_w_hbm, out_i_hbm, buf0, buf1, summary,
                   stw, sti, gids, sem):
        cid = lax.axis_index("c")
        sid = lax.axis_index("s")
        wid = sid * _NC + cid
        row0 = wid * _RPW
        iota = lax.iota(jnp.int32, _L)
        izero = jnp.zeros((_L,), jnp.int32)

        def fire_row(r, buf):
            for s in range(_SPLITS):
                pltpu.async_copy(adj_hbm.at[r, pl.ds(s * _Q, _Q)],
                                 buf.at[pl.ds(s * _Q, _Q)], sem)

        def wait_row(r, buf):
            pltpu.make_async_copy(adj_hbm.at[r], buf, sem).wait()

        def do_row(buf, lane_off):
            st = _row_topk(buf, summary, gids, iota, izero)
            return _softmax_pack(st, lane_off, iota)

        fire_row(row0, buf0)
        wait_row(row0, buf0)

        def group(g, acc):
            r0 = row0 + 4 * g
            fire_row(r0 + 1, buf1)
            w0, j0 = do_row(buf0, 0)
            wait_row(r0 + 1, buf1)

            fire_row(r0 + 2, buf0)
            w1, j1 = do_row(buf1, 4)
            wait_row(r0 + 2, buf0)

            fire_row(r0 + 3, buf1)
            w2, j2 = do_row(buf0, 8)
            wait_row(r0 + 3, buf1)

            @pl.when(g < _GROUPS - 1)
            def _():
                fire_row(r0 + 4, buf0)

            w3, j3 = do_row(buf1, 12)

            @pl.when(g < _GROUPS - 1)
            def _():
                wait_row(r0 + 4, buf0)

            stw[pl.ds(g * _L, _L)] = w0 + w1 + w2 + w3
            sti[pl.ds(g * _L, _L)] = j0 + j1 + j2 + j3
            return acc

        lax.fori_loop(0, _GROUPS, group, jnp.int32(0))

        pltpu.sync_copy(stw, out_w_hbm.at[pl.ds(row0 * _TOPK, _RPW * _TOPK)])
        pltpu.sync_copy(sti, out_i_hbm.at[pl.ds(row0 * _TOPK, _RPW * _TOPK)])

    return topk_route


_topk_route = _make_kernel()


@jax.jit
def kernel(adj):
    b, h, n = adj.shape
    w, i = _topk_route(adj.reshape(b * h, n))
    return w.reshape(b, h, _TOPK), i.reshape(b, h, _TOPK)


# speedup vs baseline: 4.1247x; 1.0552x over previous
import functools

import jax
import jax.numpy as jnp
from jax import lax
from jax.experimental import pallas as pl
from jax.experimental.pallas import tpu as pltpu
from jax.experimental.pallas import tpu_sc as plsc

_TOPK = 4
_ROWS = 1024
_COLS = 32768
_NC = 2
_NS = 16
_L = 16
_NW = _NC * _NS
_RPW = _ROWS // _NW
_GROUPS = _RPW // 4
_GRP = 32
_NGRP = _COLS // (_L * _GRP)
_SPLITS = 8
_Q = _COLS // _SPLITS
_NEG_INF = float("-inf")


def _insert(v, idx, t0, t1, t2, t3, i0, i1, i2, i3):
    m0 = v > t0
    m1 = v > t1
    m2 = v > t2
    m3 = v > t3
    t3n = jnp.where(m2, t2, jnp.where(m3, v, t3))
    i3n = jnp.where(m2, i2, jnp.where(m3, idx, i3))
    t2n = jnp.where(m1, t1, jnp.where(m2, v, t2))
    i2n = jnp.where(m1, i1, jnp.where(m2, idx, i2))
    t1n = jnp.where(m0, t0, jnp.where(m1, v, t1))
    i1n = jnp.where(m0, i0, jnp.where(m1, idx, i1))
    t0n = jnp.where(m0, v, t0)
    i0n = jnp.where(m0, idx, i0)
    return t0n, t1n, t2n, t3n, i0n, i1n, i2n, i3n


def _init_state():
    tneg = jnp.full((_L,), _NEG_INF, jnp.float32)
    izero = jnp.zeros((_L,), jnp.int32)
    return (tneg, tneg, tneg, tneg, izero, izero, izero, izero)


def _merge4(state):
    ts = list(state[:4])
    is_ = list(state[4:])
    big = jnp.int32(2**30)
    gv, gi = [], []
    for _ in range(_TOPK):
        m = jnp.maximum(jnp.maximum(ts[0], ts[1]), jnp.maximum(ts[2], ts[3]))
        gmax = jnp.max(m)
        cand = [jnp.where(tj == gmax, ij, big) for tj, ij in zip(ts, is_)]
        mn = jnp.minimum(jnp.minimum(cand[0], cand[1]),
                         jnp.minimum(cand[2], cand[3]))
        gidx = jnp.min(mn)
        gv.append(gmax)
        gi.append(gidx)
        ts = [jnp.where(ij == gidx, _NEG_INF, tj) for tj, ij in zip(ts, is_)]
    return gv, gi


def _sort4(a, b, c, d):
    a, b = jnp.minimum(a, b), jnp.maximum(a, b)
    c, d = jnp.minimum(c, d), jnp.maximum(c, d)
    a, c = jnp.minimum(a, c), jnp.maximum(a, c)
    b, d = jnp.minimum(b, d), jnp.maximum(b, d)
    b, c = jnp.minimum(b, c), jnp.maximum(b, c)
    return a, b, c, d


def _row_topk(buf, summary, gids, iota, izero):
    def phase_a(g, carry):
        base = g * (_L * _GRP)
        mx = buf[pl.ds(base, _L)]
        for u in range(1, _GRP):
            mx = jnp.maximum(mx, buf[pl.ds(base + u * _L, _L)])
        summary[pl.ds(g * _L, _L)] = mx
        t0, t1, t2, t3, i0, i1, i2, i3 = carry
        return _insert(mx, izero + g, t0, t1, t2, t3, i0, i1, i2, i3)

    sstate = lax.fori_loop(0, _NGRP, phase_a, _init_state())
    gv, gg = _merge4(sstate)
    v4 = gv[3]

    def count_body(c, acc):
        for u in range(4):
            sv = summary[pl.ds((c * 4 + u) * _L, _L)]
            acc = acc + jnp.where(sv == v4, 1, 0)
        return acc

    cvec = lax.fori_loop(0, _NGRP // 4, count_body, jnp.zeros((_L,), jnp.int32))
    cnt = jnp.sum(cvec)
    tied = cnt > 1

    g0, g1, g2, g3 = _sort4(gg[0], gg[1], gg[2], gg[3])

    @pl.when(jnp.logical_not(tied))
    def _():
        gids[0] = g0
        gids[1] = g1
        gids[2] = g2
        gids[3] = g3

    @pl.when(tied)
    def _():
        def wr(k, acc):
            gids[k] = k
            return acc
        lax.fori_loop(0, _NGRP, wr, jnp.int32(0))

    ngrp = jnp.where(tied, _NGRP, 4)

    def per_group(k, carry):
        g = gids[k]

        def chunk8(c2, carry):
            t0, t1, t2, t3, i0, i1, i2, i3 = carry
            base = g * (_L * _GRP) + c2 * (_L * 8)
            for u in range(8):
                off = base + u * _L
                v = buf[pl.ds(off, _L)]
                t0, t1, t2, t3, i0, i1, i2, i3 = _insert(
                    v, iota + off, t0, t1, t2, t3, i0, i1, i2, i3)
            return (t0, t1, t2, t3, i0, i1, i2, i3)

        return lax.fori_loop(0, _GRP // 8, chunk8, carry)

    return lax.fori_loop(0, ngrp, per_group, _init_state())


def _bcast_last(x):
    return jnp.take_along_axis(x, jnp.full((_L,), _L - 1, jnp.int32), axis=0)


def _softmax_pack(state, lane_off, iota):
    ts = list(state[:4])
    is_ = list(state[4:])
    bign = jnp.full((_L,), -(2**30), jnp.int32)
    gvs, gis = [], []
    for _ in range(_TOPK):
        m = jnp.maximum(jnp.maximum(ts[0], ts[1]), jnp.maximum(ts[2], ts[3]))
        gmax = _bcast_last(plsc.cummax(m))
        cand = [jnp.where(tj == gmax, -ij, bign) for tj, ij in zip(ts, is_)]
        mn = jnp.maximum(jnp.maximum(cand[0], cand[1]),
                         jnp.maximum(cand[2], cand[3]))
        gidx = -_bcast_last(plsc.cummax(mn))
        gvs.append(gmax)
        gis.append(gidx)
        ts = [jnp.where(ij == gidx, _NEG_INF, tj) for tj, ij in zip(ts, is_)]
    dv = jnp.zeros((_L,), jnp.float32)
    iv = jnp.zeros((_L,), jnp.int32)
    for k in range(_TOPK):
        sel = iota == (lane_off + k)
        dv = jnp.where(sel, gvs[k] - gvs[0], dv)
        iv = jnp.where(sel, gis[k], iv)
    ev = jnp.exp(dv)
    in_row = (iota >= lane_off) & (iota < lane_off + _TOPK)
    ev = jnp.where(in_row, ev, 0.0)
    wv = ev / _bcast_last(plsc.cumsum(ev))
    return wv, iv


def _make_kernel():
    mesh = plsc.VectorSubcoreMesh(core_axis_name="c", subcore_axis_name="s",
                                  num_cores=_NC, num_subcores=_NS)

    @functools.partial(
        pl.kernel,
        out_type=(
            jax.ShapeDtypeStruct((_ROWS * _TOPK,), jnp.float32),
            jax.ShapeDtypeStruct((_ROWS * _TOPK,), jnp.int32),
        ),
        mesh=mesh,
        scratch_types=(
            pltpu.VMEM((_COLS,), jnp.float32),
            pltpu.VMEM((_COLS,), jnp.float32),
            pltpu.VMEM((_NGRP * _L,), jnp.float32),
            pltpu.VMEM((_RPW * _TOPK,), jnp.float32),
            pltpu.VMEM((_RPW * _TOPK,), jnp.int32),
            pltpu.SMEM((_NGRP,), jnp.int32),
            pltpu.SemaphoreType.DMA,
        ),
        compiler_params=pltpu.CompilerParams(needs_layout_passes=False),
    )
    def topk_route(adj_hbm, out_w_hbm, out_i_hbm, buf0, buf1, summary,
                   stw, sti, gids, sem):
        cid = lax.axis_index("c")
        sid = lax.axis_index("s")
        wid = sid * _NC + cid
        row0 = wid * _RPW
        iota = lax.iota(jnp.int32, _L)
        izero = jnp.zeros((_L,), jnp.int32)

        def fire_row(r, buf):
            for s in range(_SPLITS):
                pltpu.async_copy(adj_hbm.at[r, pl.ds(s * _Q, _Q)],
                                 buf.at[pl.ds(s * _Q, _Q)], sem)

        def wait_row(r, buf):
            pltpu.make_async_copy(adj_hbm.at[r], buf, sem).wait()

        def do_row(buf, lane_off):
            st = _row_topk(buf, summary, gids, iota, izero)
            return _softmax_pack(st, lane_off, iota)

        fire_row(row0, buf0)
        wait_row(row0, buf0)

        def group(g, acc):
            r0 = row0 + 4 * g
            fire_row(r0 + 1, buf1)
            w0, j0 = do_row(buf0, 0)
            wait_row(r0 + 1, buf1)

            fire_row(r0 + 2, buf0)
            w1, j1 = do_row(buf1, 4)
            wait_row(r0 + 2, buf0)

            fire_row(r0 + 3, buf1)
            w2, j2 = do_row(buf0, 8)
            wait_row(r0 + 3, buf1)

            @pl.when(g < _GROUPS - 1)
            def _():
                fire_row(r0 + 4, buf0)

            w3, j3 = do_row(buf1, 12)

            @pl.when(g < _GROUPS - 1)
            def _():
                wait_row(r0 + 4, buf0)

            stw[pl.ds(g * _L, _L)] = w0 + w1 + w2 + w3
            sti[pl.ds(g * _L, _L)] = j0 + j1 + j2 + j3
            return acc

        lax.fori_loop(0, _GROUPS, group, jnp.int32(0))

        pltpu.sync_copy(stw, out_w_hbm.at[pl.ds(row0 * _TOPK, _RPW * _TOPK)])
        pltpu.sync_copy(sti, out_i_hbm.at[pl.ds(row0 * _TOPK, _RPW * _TOPK)])

    return topk_route


_topk_route = _make_kernel()


@jax.jit
def kernel(adj):
    b, h, n = adj.shape
    w, i = _topk_route(adj.reshape(b * h, n))
    return w.reshape(b, h, _TOPK), i.reshape(b, h, _TOPK)
